# Initial kernel scaffold; baseline (speedup 1.0000x reference)
import jax
import jax.numpy as jnp
from jax.experimental import pallas as pl


def kernel(x, edge_index, W1, b1, g1, be1, W2, b2, Wc, bc):
    raise NotImplementedError("write your pallas kernel here")


# speedup vs baseline: 14.8303x; 14.8303x over previous
import functools

import jax
import jax.numpy as jnp
from jax import lax
from jax.experimental import pallas as pl
from jax.experimental.pallas import tpu as pltpu
from jax.experimental.pallas import tpu_sc as plsc

N = 10000
E = 320000
D = 128
OUT = 70

NC = 2
NS = 16
NW = NC * NS
ET = E // NW
C = 80
NK = ET // C

NPAD = 10240
RPT = NPAD // NS
OB = 160
DC = 2000

def _zero2d(ref, rows, width):
    zero = jnp.zeros((16,), jnp.float32)

    def body(i, _):
        for j in range(width // 16):
            ref[i, pl.ds(j * 16, 16)] = zero
        return 0

    lax.fori_loop(0, rows, body, 0)


@functools.cache
def _make_deg_kernel():
    mesh = plsc.VectorSubcoreMesh(
        core_axis_name="c", subcore_axis_name="s",
        num_cores=NC, num_subcores=NS,
    )
    return pl.kernel(
        _deg_body,
        out_type=jax.ShapeDtypeStruct((NC, NS, NPAD), jnp.float32),
        mesh=mesh,
        scratch_types=[
            pltpu.VMEM((NPAD,), jnp.float32),
            pltpu.VMEM((DC,), jnp.int32),
        ],
        compiler_params=pltpu.CompilerParams(needs_layout_passes=False),
    )


def _deg_body(dst_hbm, out_hbm, cnt_v, didx_v):
    c = lax.axis_index("c")
    s = lax.axis_index("s")
    base = (c * NS + s) * ET

    zero = jnp.zeros((16,), jnp.float32)

    def zb(k, _):
        cnt_v[pl.ds(k * 16, 16)] = zero
        return 0

    lax.fori_loop(0, NPAD // 16, zb, 0)

    one = jnp.full((16,), 1.0, jnp.float32)

    def chunk(k, _):
        pltpu.sync_copy(dst_hbm.at[pl.ds(base + k * DC, DC)], didx_v)

        def inner(j, _):
            iv = didx_v[pl.ds(j * 16, 16)]
            plsc.addupdate_scatter(cnt_v, [iv], one)
            return 0

        lax.fori_loop(0, DC // 16, inner, 0)
        return 0

    lax.fori_loop(0, ET // DC, chunk, 0)
    pltpu.sync_copy(cnt_v, out_hbm.at[c, s])


@functools.cache
def _make_agg_kernel():
    mesh = plsc.VectorSubcoreMesh(
        core_axis_name="c", subcore_axis_name="s",
        num_cores=NC, num_subcores=NS,
    )
    return pl.kernel(
        _agg_body,
        out_type=jax.ShapeDtypeStruct((NC, NPAD, D), jnp.float32),
        mesh=mesh,
        scratch_types=[
            pltpu.VMEM((C,), jnp.int32),
            pltpu.VMEM((C,), jnp.int32),
            pltpu.VMEM((C, D), jnp.float32),
            pltpu.VMEM((OB, D), jnp.float32),
            pltpu.SemaphoreType.DMA,
            pltpu.VMEM_SHARED((NPAD, D), jnp.float32),
        ],
    )


def _agg_body(table_hbm, src_hbm, dst_hbm, out_hbm,
              sidx_v, didx_v, rows_v, obuf_v, sem, acc_s):
    c = lax.axis_index("c")
    s = lax.axis_index("s")
    wid = c * NS + s
    base = wid * ET

    _zero2d(obuf_v, OB, D)
    for j in range(RPT // OB):
        pltpu.sync_copy(obuf_v, acc_s.at[pl.ds(s * RPT + j * OB, OB)])
    plsc.subcore_barrier()

    def step(k, _):
        e0 = base + k * C
        pltpu.sync_copy(src_hbm.at[pl.ds(e0, C)], sidx_v)
        pltpu.sync_copy(dst_hbm.at[pl.ds(e0, C)], didx_v)
        pltpu.async_copy(table_hbm.at[sidx_v], rows_v, sem).wait()
        pltpu.sync_copy(rows_v, acc_s.at[didx_v], add=True)
        return 0

    lax.fori_loop(0, NK, step, 0)
    plsc.subcore_barrier()

    for j in range(RPT // OB):
        r0 = s * RPT + j * OB
        pltpu.sync_copy(acc_s.at[pl.ds(r0, OB)], obuf_v)
        pltpu.sync_copy(obuf_v, out_hbm.at[c, pl.ds(r0, OB)])


def _dinv_from(degp):
    deg = jnp.sum(degp.reshape(NC * NS, NPAD), axis=0)[:N, None] + 1.0
    return lax.rsqrt(deg)


def _tc1_body(x_ref, w_ref, degp_ref, g_ref):
    dinv = _dinv_from(degp_ref[...])
    m = jnp.dot(x_ref[...], w_ref[...], preferred_element_type=jnp.float32)
    g_ref[...] = dinv * m


def _tc2_body(sp_ref, g1_ref, degp_ref, b1_ref, gm_ref, bt_ref, w2_ref,
              g2_ref):
    dinv = _dinv_from(degp_ref[...])
    ssum = sp_ref[0, :N, :] + sp_ref[1, :N, :] + g1_ref[...]
    a = dinv * ssum + b1_ref[...][None, :]
    mu = jnp.mean(a, axis=0, keepdims=True)
    var = jnp.mean((a - mu) * (a - mu), axis=0, keepdims=True)
    h = (a - mu) * lax.rsqrt(var + 1e-5) * gm_ref[...][None, :]
    h = h + bt_ref[...][None, :]
    h = jnp.maximum(h, 0.0)
    m2 = jnp.dot(h, w2_ref[...], preferred_element_type=jnp.float32)
    g2_ref[...] = dinv * m2


def _tc3_body(sp_ref, g2_ref, degp_ref, b2_ref, wc_ref, bc_ref, out_ref):
    dinv = _dinv_from(degp_ref[...])
    a = dinv * (sp_ref[0, :N, :] + sp_ref[1, :N, :] + g2_ref[...])
    a = a + b2_ref[...][None, :]
    out_ref[...] = (
        jnp.dot(a, wc_ref[...], preferred_element_type=jnp.float32)
        + bc_ref[...][None, :]
    )


def kernel(x, edge_index, W1, b1, g1, be1, W2, b2, Wc, bc):
    src = edge_index[0]
    dst = edge_index[1]

    degp = _make_deg_kernel()(dst)
    agg = _make_agg_kernel()

    g1t = pl.pallas_call(
        _tc1_body, out_shape=jax.ShapeDtypeStruct((N, D), jnp.float32)
    )(x, W1, degp)

    s1p = agg(g1t, src, dst)

    g2t = pl.pallas_call(
        _tc2_body, out_shape=jax.ShapeDtypeStruct((N, D), jnp.float32)
    )(s1p, g1t, degp, b1, g1, be1, W2)

    s2p = agg(g2t, src, dst)

    out = pl.pallas_call(
        _tc3_body, out_shape=jax.ShapeDtypeStruct((N, OUT), jnp.float32)
    )(s2p, g2t, degp, b2, Wc, bc)

    return out


# speedup vs baseline: 40.7376x; 2.7469x over previous
import functools

import jax
import jax.numpy as jnp
from jax import lax
from jax.experimental import pallas as pl
from jax.experimental.pallas import tpu as pltpu
from jax.experimental.pallas import tpu_sc as plsc

N = 10000
E = 320000
D = 128
OUT = 70

NC = 2
NS = 16
NW = NC * NS
ET = E // NW
C = 80
NK = ET // C

NPAD = 10240
RPT = NPAD // NS
OB = 160
DC = 2000

def _zero2d(ref, rows, width):
    zero = jnp.zeros((16,), jnp.float32)

    def body(i, _):
        for j in range(width // 16):
            ref[i, pl.ds(j * 16, 16)] = zero
        return 0

    lax.fori_loop(0, rows, body, 0)


@functools.cache
def _make_deg_kernel():
    mesh = plsc.VectorSubcoreMesh(
        core_axis_name="c", subcore_axis_name="s",
        num_cores=NC, num_subcores=NS,
    )
    return pl.kernel(
        _deg_body,
        out_type=jax.ShapeDtypeStruct((NC, NS, NPAD), jnp.float32),
        mesh=mesh,
        scratch_types=[
            pltpu.VMEM((NPAD,), jnp.float32),
            pltpu.VMEM((DC,), jnp.int32),
        ],
        compiler_params=pltpu.CompilerParams(needs_layout_passes=False),
    )


def _deg_body(dst_hbm, out_hbm, cnt_v, didx_v):
    c = lax.axis_index("c")
    s = lax.axis_index("s")
    base = (c * NS + s) * ET

    zero = jnp.zeros((16,), jnp.float32)

    def zb(k, _):
        cnt_v[pl.ds(k * 16, 16)] = zero
        return 0

    lax.fori_loop(0, NPAD // 16, zb, 0)

    one = jnp.full((16,), 1.0, jnp.float32)

    def chunk(k, _):
        pltpu.sync_copy(dst_hbm.at[pl.ds(base + k * DC, DC)], didx_v)

        def inner(j, _):
            iv = didx_v[pl.ds(j * 16, 16)]
            plsc.addupdate_scatter(cnt_v, [iv], one)
            return 0

        lax.fori_loop(0, DC // 16, inner, 0)
        return 0

    lax.fori_loop(0, ET // DC, chunk, 0)
    pltpu.sync_copy(cnt_v, out_hbm.at[c, s])


B = 4
BI = 8
GL = 2
SL = 4
RL = 6


@functools.cache
def _make_agg_kernel():
    mesh = plsc.VectorSubcoreMesh(
        core_axis_name="c", subcore_axis_name="s",
        num_cores=NC, num_subcores=NS,
    )
    return pl.kernel(
        _agg_body,
        out_type=jax.ShapeDtypeStruct((NC, NPAD, D), jnp.float32),
        mesh=mesh,
        scratch_types=[
            pltpu.VMEM((BI, 1, C), jnp.int32),
            pltpu.VMEM((BI, 1, C), jnp.int32),
            pltpu.VMEM((B, C, D), jnp.float32),
            pltpu.SemaphoreType.DMA,
            pltpu.SemaphoreType.DMA,
            pltpu.SemaphoreType.DMA,
            pltpu.VMEM_SHARED((NPAD, D), jnp.float32),
        ],
    )


def _agg_body(table_hbm, src4_hbm, dst4_hbm, out_hbm,
              sidx_v, didx_v, rows_v, isem, gsem, ssem, acc_s):
    c = lax.axis_index("c")
    s = lax.axis_index("s")
    wid = c * NS + s

    def drain_rows(sem):
        pltpu.make_async_copy(
            table_hbm.at[pl.ds(0, C)], rows_v.at[0], sem
        ).wait()

    def drain_idx():
        pltpu.make_async_copy(src4_hbm.at[0, 0], sidx_v.at[0], isem).wait()

    _zero2d(rows_v.at[0], C, D)
    for b in range(RPT // C):
        pltpu.sync_copy(rows_v.at[0], acc_s.at[pl.ds(s * RPT + b * C, C)])
    plsc.subcore_barrier()

    def step(j, _):
        @pl.when(jnp.logical_and(j >= RL, j < NK + RL))
        def _():
            drain_rows(ssem)

        @pl.when(j < NK)
        def _():
            islot = lax.rem(j, BI)
            pltpu.async_copy(src4_hbm.at[wid, j], sidx_v.at[islot], isem)
            pltpu.async_copy(dst4_hbm.at[wid, j], didx_v.at[islot], isem)

        @pl.when(jnp.logical_and(j >= GL, j < NK + GL))
        def _():
            k = j - GL
            drain_idx()
            drain_idx()
            pltpu.async_copy(
                table_hbm.at[sidx_v.at[lax.rem(k, BI), 0]],
                rows_v.at[lax.rem(k, B)], gsem)

        @pl.when(jnp.logical_and(j >= SL, j < NK + SL))
        def _():
            k = j - SL
            drain_rows(gsem)
            pltpu.async_copy(
                rows_v.at[lax.rem(k, B)],
                acc_s.at[didx_v.at[lax.rem(k, BI), 0]], ssem, add=True)
        return 0

    lax.fori_loop(0, NK + RL, step, 0)
    plsc.subcore_barrier()

    nb = RPT // C
    for b in range(B):
        pltpu.async_copy(acc_s.at[pl.ds(s * RPT + b * C, C)], rows_v.at[b],
                         gsem)
    for b in range(nb):
        drain_rows(gsem)
        pltpu.async_copy(rows_v.at[b % B],
                         out_hbm.at[c, pl.ds(s * RPT + b * C, C)], ssem)
        drain_rows(ssem)
        if b + B < nb:
            pltpu.async_copy(acc_s.at[pl.ds(s * RPT + (b + B) * C, C)],
                             rows_v.at[b % B], gsem)


def _dinv_from(degp):
    deg = jnp.sum(degp.reshape(NC * NS, NPAD), axis=0)[:N, None] + 1.0
    return lax.rsqrt(deg)


def _tc1_body(x_ref, w_ref, degp_ref, g_ref):
    dinv = _dinv_from(degp_ref[...])
    m = jnp.dot(x_ref[...], w_ref[...], preferred_element_type=jnp.float32)
    g_ref[...] = dinv * m


def _tc2_body(sp_ref, g1_ref, degp_ref, b1_ref, gm_ref, bt_ref, w2_ref,
              g2_ref):
    dinv = _dinv_from(degp_ref[...])
    ssum = sp_ref[0, :N, :] + sp_ref[1, :N, :] + g1_ref[...]
    a = dinv * ssum + b1_ref[...][None, :]
    mu = jnp.mean(a, axis=0, keepdims=True)
    var = jnp.mean((a - mu) * (a - mu), axis=0, keepdims=True)
    h = (a - mu) * lax.rsqrt(var + 1e-5) * gm_ref[...][None, :]
    h = h + bt_ref[...][None, :]
    h = jnp.maximum(h, 0.0)
    m2 = jnp.dot(h, w2_ref[...], preferred_element_type=jnp.float32)
    g2_ref[...] = dinv * m2


def _tc3_body(sp_ref, g2_ref, degp_ref, b2_ref, wc_ref, bc_ref, out_ref):
    dinv = _dinv_from(degp_ref[...])
    a = dinv * (sp_ref[0, :N, :] + sp_ref[1, :N, :] + g2_ref[...])
    a = a + b2_ref[...][None, :]
    out_ref[...] = (
        jnp.dot(a, wc_ref[...], preferred_element_type=jnp.float32)
        + bc_ref[...][None, :]
    )


def kernel(x, edge_index, W1, b1, g1, be1, W2, b2, Wc, bc):
    src = edge_index[0]
    dst = edge_index[1]
    src2 = src.reshape(NW, NK, 1, C)
    dst2 = dst.reshape(NW, NK, 1, C)

    degp = _make_deg_kernel()(dst)
    agg = _make_agg_kernel()

    g1t = pl.pallas_call(
        _tc1_body, out_shape=jax.ShapeDtypeStruct((N, D), jnp.float32)
    )(x, W1, degp)

    s1p = agg(g1t, src2, dst2)

    g2t = pl.pallas_call(
        _tc2_body, out_shape=jax.ShapeDtypeStruct((N, D), jnp.float32)
    )(s1p, g1t, degp, b1, g1, be1, W2)

    s2p = agg(g2t, src2, dst2)

    out = pl.pallas_call(
        _tc3_body, out_shape=jax.ShapeDtypeStruct((N, OUT), jnp.float32)
    )(s2p, g2t, degp, b2, Wc, bc)

    return out
